# Initial kernel scaffold; baseline (speedup 1.0000x reference)
import jax
import jax.numpy as jnp
from jax.experimental import pallas as pl


def kernel(xyz, params):
    raise NotImplementedError("write your pallas kernel here")


# speedup vs baseline: 2.3665x; 2.3665x over previous
import functools

import jax
import jax.numpy as jnp
from jax.experimental import pallas as pl

_F32 = jnp.float32
_EPS = 1e-5
_DEF = jax.lax.Precision.DEFAULT
_HI = jax.lax.Precision.HIGHEST


def _iota(n, dtype=jnp.int32):
    return jax.lax.broadcasted_iota(dtype, (1, n), 1)


def _bn_relu(x, bn4):
    mean, denom, gamma, beta = bn4
    t = x - mean
    t = t / denom
    t = t * gamma
    t = t + beta
    return jnp.maximum(t, 0.0)


def _fps_body(x_ref, y_ref, z_ref, c0_ref, out_ref, *, npoint):
    X = x_ref[...]
    Y = y_ref[...]
    Z = z_ref[...]
    b, n = X.shape
    iota = _iota(n)
    iota_s = _iota(npoint)

    def body(i, st):
        dist_min, far, cents = st
        oh_s = (iota_s == i).astype(jnp.int32)
        cents = cents * (1 - oh_s) + far * oh_s
        oh = (iota == far).astype(_F32)
        cx = jnp.sum(X * oh, axis=1, keepdims=True)
        cy = jnp.sum(Y * oh, axis=1, keepdims=True)
        cz = jnp.sum(Z * oh, axis=1, keepdims=True)
        dx = X - cx
        dy = Y - cy
        dz = Z - cz
        d = (dx * dx + dy * dy) + dz * dz
        dist_min = jnp.minimum(dist_min, d)
        m = jnp.max(dist_min, axis=1, keepdims=True)
        far = jnp.min(jnp.where(dist_min == m, iota, n), axis=1, keepdims=True)
        return dist_min, far, cents

    st = (jnp.full((b, n), 1e10, _F32), jnp.zeros((b, 1), jnp.int32),
          c0_ref[...])
    _, _, cents = jax.lax.fori_loop(0, npoint, body, st)
    out_ref[...] = cents


def _fps(xyz_l, npoint):
    b, _, n = xyz_l.shape
    return pl.pallas_call(
        functools.partial(_fps_body, npoint=npoint),
        out_shape=jax.ShapeDtypeStruct((b, npoint), jnp.int32),
    )(xyz_l[:, 0, :], xyz_l[:, 1, :], xyz_l[:, 2, :],
      jnp.zeros((b, npoint), jnp.int32))


def _group_body(*refs, K, r2, S, bn):
    it = iter(refs)
    idx_ref = next(it)
    xyzr_ref = next(it)
    xyzl_ref = next(it)
    feat_ref = next(it)
    bn_refs = [next(it) for _ in range(4)] if bn else None
    nxyz_ref = next(it)
    gall_ref = next(it)

    n = xyzr_ref.shape[1]
    idx = idx_ref[0, 0]
    iota = _iota(n)
    oh = (idx == iota).astype(_F32)
    xl = xyzl_ref[0]
    xr = xyzr_ref[0]
    xrow = xl[0:1]
    yrow = xl[1:2]
    zrow = xl[2:3]
    nx = jnp.sum(oh * xrow, axis=1, keepdims=True)
    ny = jnp.sum(oh * yrow, axis=1, keepdims=True)
    nz = jnp.sum(oh * zrow, axis=1, keepdims=True)
    new_xyz = jnp.concatenate([nx, ny, nz], axis=1)
    mm = jax.lax.dot_general(new_xyz, xr, (((1,), (1,)), ((), ())),
                             preferred_element_type=_F32, precision=_DEF)
    src2 = (nx * nx + ny * ny) + nz * nz
    dst2 = (xrow * xrow + yrow * yrow) + zrow * zrow
    d = -2.0 * mm
    d = d + src2
    d = d + dst2
    mask = (d <= r2).astype(_F32)
    c = mask
    shift = 1
    while shift < n:
        rolled = jnp.concatenate([c[:, n - shift:], c[:, :n - shift]], axis=1)
        c = c + jnp.where(iota >= shift, rolled, 0.0)
        shift *= 2
    cp = c * mask
    cnt = c[:, n - 1:n]
    feats = feat_ref[0]
    if bn:
        feats = _bn_relu(feats, [r[...] for r in bn_refs])
    nxyz_ref[...] = new_xyz[None]
    lastcol = (iota == (n - 1)).astype(_F32)
    zerofb = (cnt <= 0.0).astype(_F32)
    g0 = None
    for k in range(K):
        selk = (cp == float(k + 1)).astype(_F32)
        if k == 0:
            selk = selk + zerofb * lastcol
        gf = jax.lax.dot_general(selk, feats, (((1,), (0,)), ((), ())),
                                 preferred_element_type=_F32, precision=_HI)
        gx = jax.lax.dot_general(selk, xr, (((1,), (0,)), ((), ())),
                                 preferred_element_type=_F32, precision=_HI)
        g = jnp.concatenate([gx, gf], axis=1)
        if k == 0:
            g0 = g
        else:
            g = jnp.where(cnt <= float(k), g0, g)
        g = jnp.concatenate([g[:, :3] - new_xyz, g[:, 3:]], axis=1)
        gall_ref[0, k * S:(k + 1) * S, :] = g


def _group(fps_idx, xyz_r, xyz_l, feats, bn4, radius, K, S):
    b, n, _ = xyz_r.shape
    s = fps_idx.shape[1]
    C = feats.shape[2]
    ns = s // S
    idx4 = fps_idx.reshape(b, ns, S, 1)
    in_specs = [
        pl.BlockSpec((1, 1, S, 1), lambda i, j: (i, j, 0, 0)),
        pl.BlockSpec((1, n, 3), lambda i, j: (i, 0, 0)),
        pl.BlockSpec((1, 3, n), lambda i, j: (i, 0, 0)),
        pl.BlockSpec((1, n, C), lambda i, j: (i, 0, 0)),
    ]
    args = [idx4, xyz_r, xyz_l, feats]
    if bn4 is not None:
        for v in bn4:
            args.append(v.reshape(1, C))
            in_specs.append(pl.BlockSpec((1, C), lambda i, j: (0, 0)))
    out_shape = (jax.ShapeDtypeStruct((b, s, 3), _F32),
                 jax.ShapeDtypeStruct((b, s * K, 3 + C), _F32))
    out_specs = (pl.BlockSpec((1, S, 3), lambda i, j: (i, j, 0)),
                 pl.BlockSpec((1, S * K, 3 + C), lambda i, j: (i, j, 0)))
    body = functools.partial(_group_body, K=K, r2=float(radius) ** 2, S=S,
                             bn=bn4 is not None)
    return pl.pallas_call(body, grid=(b, ns), in_specs=in_specs,
                          out_specs=out_specs, out_shape=out_shape)(*args)


def _conv(x, bn4, W, bias, Mb=2048):
    M, Cin = x.shape
    Mb = min(M, Mb)
    Cout = bias.shape[0]
    args = [x]
    in_specs = [pl.BlockSpec((Mb, Cin), lambda i: (i, 0))]
    if bn4 is not None:
        for v in bn4:
            args.append(v.reshape(1, Cin))
            in_specs.append(pl.BlockSpec((1, Cin), lambda i: (0, 0)))
    args += [W, bias.reshape(1, Cout)]
    in_specs += [pl.BlockSpec(W.shape, lambda i: (0, 0)),
                 pl.BlockSpec((1, Cout), lambda i: (0, 0))]

    def body(*refs):
        it = iter(refs)
        xv = next(it)[...]
        if bn4 is not None:
            xv = _bn_relu(xv, [next(it)[...] for _ in range(4)])
        Wv = next(it)[...]
        bv = next(it)[...]
        y_ref = next(it)
        y = jax.lax.dot_general(xv, Wv, (((1,), (1,)), ((), ())),
                                preferred_element_type=_F32, precision=_DEF)
        y_ref[...] = y + bv

    return pl.pallas_call(
        body, grid=(M // Mb,), in_specs=in_specs,
        out_specs=pl.BlockSpec((Mb, Cout), lambda i: (i, 0)),
        out_shape=jax.ShapeDtypeStruct((M, Cout), _F32))(*args)


def _bn4_from(mean, var, blk):
    return (mean, jnp.sqrt(var + _EPS), blk['gamma'], blk['beta'])


def _sa_stats(y, b, ns, K, S, blk):
    C = y.shape[1]
    yt = y.reshape(b, ns, K, S, C).transpose(0, 4, 2, 1, 3).reshape(b, C, K, ns * S)
    yt = jax.lax.optimization_barrier(yt)
    return _bn4_from(jnp.mean(yt, axis=(0, 2, 3)), jnp.var(yt, axis=(0, 2, 3)), blk)


def _fp_stats(y, b, n, blk):
    C = y.shape[1]
    yt = y.reshape(b, n, C).transpose(0, 2, 1)
    yt = jax.lax.optimization_barrier(yt)
    return _bn4_from(jnp.mean(yt, axis=(0, 2)), jnp.var(yt, axis=(0, 2)), blk)


def _maxpool(y, rows, K, S):
    C = y.shape[1]

    def body(y_ref, o_ref):
        o_ref[...] = jnp.max(y_ref[...].reshape(K, S, C), axis=0)

    return pl.pallas_call(
        body, grid=(rows // S,),
        in_specs=[pl.BlockSpec((S * K, C), lambda i: (i, 0))],
        out_specs=pl.BlockSpec((S, C), lambda i: (i, 0)),
        out_shape=jax.ShapeDtypeStruct((rows, C), _F32))(y)


def _interp_body(*refs, n2, with_p1):
    it = iter(refs)
    x1_ref = next(it)
    x2r_ref = next(it)
    x2l_ref = next(it)
    y2_ref = next(it)
    bn2_refs = [next(it) for _ in range(4)]
    if with_p1:
        y1_ref = next(it)
        bn1_refs = [next(it) for _ in range(4)]
    o_ref = next(it)
    if with_p1:
        o1_ref = next(it)

    x1 = x1_ref[0]
    x2r = x2r_ref[0]
    xl = x2l_ref[0]
    mm = jax.lax.dot_general(x1, x2r, (((1,), (1,)), ((), ())),
                             preferred_element_type=_F32, precision=_DEF)
    sq = x1 * x1
    s1 = (sq[:, 0:1] + sq[:, 1:2]) + sq[:, 2:3]
    xrow = xl[0:1]
    yrow = xl[1:2]
    zrow = xl[2:3]
    s2 = (xrow * xrow + yrow * yrow) + zrow * zrow
    d = -2.0 * mm
    d = d + s1
    d = d + s2
    iota = _iota(n2)
    vals, idxs = [], []
    dd = d
    for _ in range(3):
        m = jnp.min(dd, axis=1, keepdims=True)
        i = jnp.min(jnp.where(dd == m, iota, n2), axis=1, keepdims=True)
        vals.append(m)
        idxs.append(i)
        dd = jnp.where(iota == i, 1e30, dd)
    r0 = 1.0 / (vals[0] + 1e-8)
    r1 = 1.0 / (vals[1] + 1e-8)
    r2_ = 1.0 / (vals[2] + 1e-8)
    ssum = (r0 + r1) + r2_
    Wm = ((r0 / ssum) * (iota == idxs[0]).astype(_F32)
          + (r1 / ssum) * (iota == idxs[1]).astype(_F32)
          + (r2_ / ssum) * (iota == idxs[2]).astype(_F32))
    p2 = _bn_relu(y2_ref[0], [r[...] for r in bn2_refs])
    o_ref[...] = jax.lax.dot_general(Wm, p2, (((1,), (0,)), ((), ())),
                                     preferred_element_type=_F32,
                                     precision=_HI)[None]
    if with_p1:
        o1_ref[...] = _bn_relu(y1_ref[0], [r[...] for r in bn1_refs])[None]


def _interp(x1r, x2r, x2l, y2, bn2, y1, bn1, Nb=1024):
    b, n1, _ = x1r.shape
    n2 = x2r.shape[1]
    C2 = y2.shape[2]
    Nb = min(n1, Nb)
    args = [x1r, x2r, x2l, y2]
    in_specs = [pl.BlockSpec((1, Nb, 3), lambda i, j: (i, j, 0)),
                pl.BlockSpec((1, n2, 3), lambda i, j: (i, 0, 0)),
                pl.BlockSpec((1, 3, n2), lambda i, j: (i, 0, 0)),
                pl.BlockSpec((1, n2, C2), lambda i, j: (i, 0, 0))]
    for v in bn2:
        args.append(v.reshape(1, C2))
        in_specs.append(pl.BlockSpec((1, C2), lambda i, j: (0, 0)))
    with_p1 = y1 is not None
    if with_p1:
        C1 = y1.shape[2]
        args.append(y1)
        in_specs.append(pl.BlockSpec((1, Nb, C1), lambda i, j: (i, j, 0)))
        for v in bn1:
            args.append(v.reshape(1, C1))
            in_specs.append(pl.BlockSpec((1, C1), lambda i, j: (0, 0)))
        out_shape = (jax.ShapeDtypeStruct((b, n1, C2), _F32),
                     jax.ShapeDtypeStruct((b, n1, C1), _F32))
        out_specs = (pl.BlockSpec((1, Nb, C2), lambda i, j: (i, j, 0)),
                     pl.BlockSpec((1, Nb, C1), lambda i, j: (i, j, 0)))
    else:
        out_shape = jax.ShapeDtypeStruct((b, n1, C2), _F32)
        out_specs = pl.BlockSpec((1, Nb, C2), lambda i, j: (i, j, 0))
    return pl.pallas_call(
        functools.partial(_interp_body, n2=n2, with_p1=with_p1),
        grid=(b, n1 // Nb), in_specs=in_specs,
        out_specs=out_specs, out_shape=out_shape)(*args)


def _l2norm(y, bn4, Mb=2048):
    M, C = y.shape

    def body(y_ref, m_ref, d_ref, g_ref, b_ref, o_ref):
        x0 = _bn_relu(y_ref[...], (m_ref[...], d_ref[...], g_ref[...], b_ref[...]))
        nrm = jnp.sqrt(jnp.sum(x0 * x0, axis=1, keepdims=True))
        o_ref[...] = x0 / jnp.maximum(nrm, 1e-12)

    return pl.pallas_call(
        body, grid=(M // Mb,),
        in_specs=[pl.BlockSpec((Mb, C), lambda i: (i, 0))]
        + [pl.BlockSpec((1, C), lambda i: (0, 0))] * 4,
        out_specs=pl.BlockSpec((Mb, C), lambda i: (i, 0)),
        out_shape=jax.ShapeDtypeStruct((M, C), _F32),
    )(y, *[v.reshape(1, C) for v in bn4])


def _logits(yh, bn4, W, bvec, Mb=2048):
    M, C = yh.shape
    O = W.shape[0]

    def body(y_ref, m_ref, d_ref, g_ref, b_ref, w_ref, bias_ref, o_ref):
        x = _bn_relu(y_ref[...], (m_ref[...], d_ref[...], g_ref[...], b_ref[...]))
        z = jax.lax.dot_general(x, w_ref[...], (((1,), (1,)), ((), ())),
                                preferred_element_type=_F32,
                                precision=_DEF) + bias_ref[...]
        m = jnp.max(z, axis=1, keepdims=True)
        e = z - m
        lse = jnp.log(jnp.sum(jnp.exp(e), axis=1, keepdims=True))
        o_ref[...] = e - lse

    return pl.pallas_call(
        body, grid=(M // Mb,),
        in_specs=[pl.BlockSpec((Mb, C), lambda i: (i, 0))]
        + [pl.BlockSpec((1, C), lambda i: (0, 0))] * 4
        + [pl.BlockSpec((O, C), lambda i: (0, 0)),
           pl.BlockSpec((1, O), lambda i: (0, 0))],
        out_specs=pl.BlockSpec((Mb, O), lambda i: (i, 0)),
        out_shape=jax.ShapeDtypeStruct((M, O), _F32),
    )(yh, *[v.reshape(1, C) for v in bn4], W, bvec.reshape(1, O))


def _sa_stage(b, xyz_r, xyz_l, feats, bn_in, npoint, radius, K, S, blocks):
    fi = _fps(xyz_l, npoint)
    nxyz, g_all = _group(fi, xyz_r, xyz_l, feats, bn_in, radius, K, S)
    M = b * npoint * K
    ns = npoint // S
    blk = blocks[0]
    y = _conv(g_all.reshape(M, -1), None, blk['W'], blk['b'])
    bn4 = _sa_stats(y, b, ns, K, S, blk)
    for blk in blocks[1:]:
        y = _conv(y, bn4, blk['W'], blk['b'])
        bn4 = _sa_stats(y, b, ns, K, S, blk)
    ym = _maxpool(y, b * npoint, K, S)
    return nxyz, jnp.transpose(nxyz, (0, 2, 1)), ym.reshape(b, npoint, -1), bn4


def _fp_stage(b, x1r, x2r, x2l, y1, bn1, y2, bn2, blocks):
    n1 = x1r.shape[1]
    M = b * n1
    blk = blocks[0]
    if y1 is not None:
        interp, x1a = _interp(x1r, x2r, x2l, y2, bn2, y1, bn1)
        C1 = y1.shape[2]
        xcat = jnp.concatenate([x1a.reshape(M, C1),
                                interp.reshape(M, -1)], axis=1)
    else:
        interp = _interp(x1r, x2r, x2l, y2, bn2, None, None)
        xcat = interp.reshape(M, -1)
    y = _conv(xcat, None, blk['W'], blk['b'])
    bn4 = _fp_stats(y, b, n1, blk)
    for blk in blocks[1:]:
        y = _conv(y, bn4, blk['W'], blk['b'])
        bn4 = _fp_stats(y, b, n1, blk)
    return y.reshape(b, n1, -1), bn4


def kernel(xyz, params):
    b, _, n = xyz.shape
    xyz_r9 = jnp.transpose(xyz, (0, 2, 1))
    l0_xyz_r = xyz_r9[:, :, :3]
    l0_xyz_l = xyz[:, :3, :]

    l1r, l1l, y1, b1 = _sa_stage(b, l0_xyz_r, l0_xyz_l, xyz_r9, None,
                                 1024, 0.1, 32, 32, params['sa1'])
    l2r, l2l, y2, b2 = _sa_stage(b, l1r, l1l, y1, b1,
                                 256, 0.2, 32, 64, params['sa2'])
    l3r, l3l, y3, b3 = _sa_stage(b, l2r, l2l, y2, b2,
                                 64, 0.4, 32, 64, params['sa3'])
    l4r, l4l, y4, b4 = _sa_stage(b, l3r, l3l, y3, b3,
                                 16, 0.8, 32, 16, params['sa4'])

    f3, a3 = _fp_stage(b, l3r, l4r, l4l, y3, b3, y4, b4, params['fp4'])
    f2, a2 = _fp_stage(b, l2r, l3r, l3l, y2, b2, f3, a3, params['fp3'])
    f1, a1 = _fp_stage(b, l1r, l2r, l2l, y1, b1, f2, a2, params['fp2'])
    f0, a0 = _fp_stage(b, l0_xyz_r, l1r, l1l, None, None, f1, a1,
                       params['fp1'])

    M = b * n
    y_f0 = f0.reshape(M, -1)
    l0n = _l2norm(y_f0, a0)
    blk = params['head1']
    yh = _conv(y_f0, a0, blk['W'], blk['b'])
    ah = _fp_stats(yh, b, n, blk)
    z = _logits(yh, ah, params['conv2']['W'], params['conv2']['b'])

    x_out = z.reshape(b, n, -1)
    l0n_out = jnp.transpose(l0n.reshape(b, n, -1), (0, 2, 1))
    return (x_out, l0n_out)
